# Initial kernel scaffold; baseline (speedup 1.0000x reference)
import jax
import jax.numpy as jnp
from jax.experimental import pallas as pl


def kernel(h, peaks, labels):
    raise NotImplementedError("write your pallas kernel here")


# speedup vs baseline: 3.8109x; 3.8109x over previous
import jax
import jax.numpy as jnp
from jax.experimental import pallas as pl
import jax.experimental.pallas.tpu as pltpu

_K = 10
_TAU = 0.07
_Q, _P, _D, _C = 4096, 8192, 768, 1000
_CP = 1024
_BQ = 512
_BP = 1024
_NPJ = _P // _BP
_BIG = 2 ** 30


def _topk_block(x, gidx, neg):
    vals, idxs = [], []
    for _ in range(_K):
        m = jnp.max(x, axis=-1, keepdims=True)
        loc = jnp.min(jnp.where(x == m, gidx, _BIG), axis=-1, keepdims=True)
        x = jnp.where(gidx == loc, neg, x)
        vals.append(m)
        idxs.append(loc)
    return jnp.concatenate(vals, axis=-1), jnp.concatenate(idxs, axis=-1)


def _body(h_ref, peaks_ref, labels_ref, pi_ref, dens_ref, idx_ref,
          hn_ref, den_ref, tv_ref, ti_ref):
    pj = pl.program_id(1)
    neg = jnp.float32(-jnp.inf)

    @pl.when(pj == 0)
    def _():
        hb = h_ref[...]
        hn_ref[...] = hb / jnp.sqrt(jnp.sum(hb * hb, axis=-1, keepdims=True))

    pb = peaks_ref[...]
    pn = pb / jnp.sqrt(jnp.sum(pb * pb, axis=-1, keepdims=True))
    sim = jax.lax.dot_general(hn_ref[...], pn, (((1,), (1,)), ((), ())),
                              preferred_element_type=jnp.float32)
    e = jnp.exp(sim / _TAU)
    num = jax.lax.dot_general(e.astype(jnp.bfloat16), labels_ref[...],
                              (((1,), (0,)), ((), ())),
                              preferred_element_type=jnp.float32)
    ds = jnp.sum(e, axis=-1, keepdims=True)

    gidx = jax.lax.broadcasted_iota(jnp.int32, (_BQ, _BP), 1) + pj * _BP
    bv, bi = _topk_block(sim, gidx, neg)

    @pl.when(pj == 0)
    def _():
        pi_ref[...] = num
        den_ref[...] = ds
        tv_ref[:, :_K] = bv
        ti_ref[:, :_K] = bi
        tv_ref[:, _K:] = jnp.full((_BQ, 16 - _K), neg, jnp.float32)
        ti_ref[:, _K:] = jnp.full((_BQ, 16 - _K), _BIG, jnp.int32)

    @pl.when(pj > 0)
    def _():
        pi_ref[...] += num
        den_ref[...] += ds
        cv = jnp.concatenate([tv_ref[...], bv], axis=-1)
        ci = jnp.concatenate([ti_ref[...], bi], axis=-1)
        mv, mi = _topk_block(cv, ci, neg)
        tv_ref[:, :_K] = mv
        ti_ref[:, :_K] = mi

    @pl.when(pj == _NPJ - 1)
    def _():
        pi_ref[...] = pi_ref[...] / den_ref[...]
        dens_ref[...] = jnp.sum(tv_ref[:, :_K], axis=-1,
                                keepdims=True) * jnp.float32(1.0 / _K)
        idx_ref[...] = ti_ref[...]


def kernel(h, peaks, labels):
    labels_p = jnp.pad(labels, ((0, 0), (0, _CP - _C))).astype(jnp.bfloat16)
    grid = (_Q // _BQ, _NPJ)
    pi_p, dens, idx = pl.pallas_call(
        _body,
        grid=grid,
        in_specs=[
            pl.BlockSpec((_BQ, _D), lambda qi, pj: (qi, 0)),
            pl.BlockSpec((_BP, _D), lambda qi, pj: (pj, 0)),
            pl.BlockSpec((_BP, _CP), lambda qi, pj: (pj, 0)),
        ],
        out_specs=[
            pl.BlockSpec((_BQ, _CP), lambda qi, pj: (qi, 0)),
            pl.BlockSpec((_BQ, 1), lambda qi, pj: (qi, 0)),
            pl.BlockSpec((_BQ, 16), lambda qi, pj: (qi, 0)),
        ],
        out_shape=[
            jax.ShapeDtypeStruct((_Q, _CP), jnp.float32),
            jax.ShapeDtypeStruct((_Q, 1), jnp.float32),
            jax.ShapeDtypeStruct((_Q, 16), jnp.int32),
        ],
        scratch_shapes=[
            pltpu.VMEM((_BQ, _D), jnp.float32),
            pltpu.VMEM((_BQ, 1), jnp.float32),
            pltpu.VMEM((_BQ, 16), jnp.float32),
            pltpu.VMEM((_BQ, 16), jnp.int32),
        ],
    )(h, peaks, labels_p)
    return pi_p[:, :_C], dens[:, 0], idx[:, :_K]
